# baseline (speedup 1.0000x reference)
import jax
import jax.numpy as jnp
from jax.experimental import pallas as pl
from jax.experimental.pallas import tpu as pltpu

ROWS = 64
VOCAB = 1_000_000
L = 8_192
NBLK = -(-VOCAB // L)
CHUNK = 1_024
CPB = L // CHUNK
LASTCHUNK = (VOCAB - 1) // CHUNK


def _sums_body(p_ref, out_ref):
    b = pl.program_id(0)
    x = p_ref[:, :]
    lane = jax.lax.broadcasted_iota(jnp.int32, (ROWS, L), 1) + b * L
    x = jnp.where(lane < VOCAB, x, 0.0)
    parts = [jnp.sum(x[:, c * CHUNK:(c + 1) * CHUNK], axis=1, keepdims=True)
             for c in range(CPB)]
    out_ref[0, :, :] = jnp.concatenate(parts, axis=1)


def _scan_body(s3_ref, l123_ref, l8_ref, rngr_ref, cb2_ref, thr2_ref):
    s3 = s3_ref[:, :, :]
    bt = jnp.sum(s3, axis=2)
    btcs = jnp.dot(l123_ref[:, :], bt, preferred_element_type=jnp.float32)
    rng = rngr_ref[:, :]
    below_b = btcs < rng
    cb = jnp.minimum(jnp.sum(below_b.astype(jnp.int32), axis=0, keepdims=True),
                     NBLK - 1)
    bidx = jax.lax.broadcasted_iota(jnp.int32, (NBLK, ROWS), 0)
    bp = jnp.max(jnp.where(bidx < cb, btcs, 0.0), axis=0, keepdims=True)
    thr1 = rng - bp

    cols = [jnp.sum(jnp.where(bidx == cb, s3[:, :, c], 0.0), axis=0,
                    keepdims=True) for c in range(CPB)]
    m = jnp.concatenate(cols, axis=0)
    mcs = jnp.dot(l8_ref[:, :], m, preferred_element_type=jnp.float32)
    below_c = mcs < thr1
    ch = jnp.sum(below_c.astype(jnp.int32), axis=0, keepdims=True)
    ch = jnp.minimum(ch, jnp.minimum(CPB - 1, LASTCHUNK - cb * CPB))
    cidx = jax.lax.broadcasted_iota(jnp.int32, (CPB, ROWS), 0)
    cp = jnp.max(jnp.where(cidx < ch, mcs, 0.0), axis=0, keepdims=True)
    cb2_ref[:, :] = cb * CPB + ch
    thr2_ref[:, :] = thr1 - cp


def _finish_body(cb2_smem, p_ref, tri_ref, thr_ref, out_ref):
    r = pl.program_id(0)
    q = r - 8 * (r // 8)
    g = p_ref[:, :]
    csum = jnp.dot(g, tri_ref[:, :], preferred_element_type=jnp.float32)
    glane = (jax.lax.broadcasted_iota(jnp.int32, (8, CHUNK), 1)
             + cb2_smem[r] * CHUNK)
    below = (csum < thr_ref[:, :]) & (glane < VOCAB)
    cnt8 = jnp.sum(below.astype(jnp.int32), axis=1, keepdims=True)
    sub = jax.lax.broadcasted_iota(jnp.int32, (8, 1), 0)
    cnt = jnp.sum(jnp.where(sub == q, cnt8, 0), axis=0, keepdims=True)
    out_ref[0, :, :] = cb2_smem[r] * CHUNK + cnt


def _tri(n, lower):
    a = jax.lax.broadcasted_iota(jnp.int32, (n, n), 0)
    b = jax.lax.broadcasted_iota(jnp.int32, (n, n), 1)
    return ((b <= a) if lower else (a <= b)).astype(jnp.float32)


def kernel(p):
    rng = jax.random.uniform(jax.random.key(0), (ROWS,), dtype=jnp.float32)

    s3 = pl.pallas_call(
        _sums_body,
        grid=(NBLK,),
        in_specs=[pl.BlockSpec((ROWS, L), lambda b: (0, b))],
        out_specs=pl.BlockSpec((1, ROWS, CPB), lambda b: (b, 0, 0)),
        out_shape=jax.ShapeDtypeStruct((NBLK, ROWS, CPB), jnp.float32),
    )(p)

    cb2, thr2 = pl.pallas_call(
        _scan_body,
        in_specs=[
            pl.BlockSpec((NBLK, ROWS, CPB), lambda: (0, 0, 0)),
            pl.BlockSpec((NBLK, NBLK), lambda: (0, 0)),
            pl.BlockSpec((CPB, CPB), lambda: (0, 0)),
            pl.BlockSpec((1, ROWS), lambda: (0, 0)),
        ],
        out_specs=[
            pl.BlockSpec((1, ROWS), lambda: (0, 0)),
            pl.BlockSpec((1, ROWS), lambda: (0, 0)),
        ],
        out_shape=[
            jax.ShapeDtypeStruct((1, ROWS), jnp.int32),
            jax.ShapeDtypeStruct((1, ROWS), jnp.float32),
        ],
    )(s3, _tri(NBLK, True), _tri(CPB, True), rng.reshape(1, ROWS))

    out = pl.pallas_call(
        _finish_body,
        grid_spec=pltpu.PrefetchScalarGridSpec(
            num_scalar_prefetch=1,
            grid=(ROWS,),
            in_specs=[
                pl.BlockSpec((8, CHUNK), lambda r, cb2_s: (r // 8, cb2_s[r])),
                pl.BlockSpec((CHUNK, CHUNK), lambda r, cb2_s: (0, 0)),
                pl.BlockSpec((8, 1), lambda r, cb2_s: (r // 8, 0)),
            ],
            out_specs=pl.BlockSpec((1, 1, 1), lambda r, cb2_s: (r, 0, 0)),
        ),
        out_shape=jax.ShapeDtypeStruct((ROWS, 1, 1), jnp.int32),
    )(cb2.reshape(ROWS), p, _tri(CHUNK, False), thr2.reshape(ROWS, 1))

    return jax.lax.stop_gradient(out.reshape(ROWS, 1))


# speedup vs baseline: 1.2206x; 1.2206x over previous
import functools

import jax
import jax.numpy as jnp
from jax import lax
from jax.experimental import pallas as pl
from jax.experimental.pallas import tpu as pltpu
from jax.experimental.pallas import tpu_sc as plsc

ROWS = 64
VOCAB = 1_000_000
L = 8_192
NBLK = -(-VOCAB // L)
CHUNK = 1_024
CPB = L // CHUNK
NCHUNK = NBLK * CPB
NCH_PAD = 1_024
LASTCHUNK = (VOCAB - 1) // CHUNK
WIN = CHUNK + 128
WSTART_MAX = 998_912
NW = 32
RPW = ROWS // NW


def _sums_body(p_ref, out_ref):
    b = pl.program_id(0)
    x = p_ref[:, :]
    lane = jax.lax.broadcasted_iota(jnp.int32, (ROWS, L), 1) + b * L
    x = jnp.where(lane < VOCAB, x, 0.0)
    parts = [jnp.sum(x[:, c * CHUNK:(c + 1) * CHUNK], axis=1, keepdims=True)
             for c in range(CPB)]
    out_ref[0, :, :] = jnp.concatenate(parts, axis=1)


def _sc_finish(sums_hbm, rng_hbm, p_hbm, out_hbm, sums_v, win_v, rng_v,
               out_v):
    cid = lax.axis_index("c")
    sid = lax.axis_index("s")
    wid = sid * 2 + cid
    iota16 = lax.iota(jnp.int32, 16)

    for k in range(RPW):
        r = wid * RPW + k
        pltpu.sync_copy(sums_hbm.at[pl.ds(r * NCH_PAD, NCH_PAD)], sums_v)
        pltpu.sync_copy(rng_hbm.at[pl.ds(r * 16, 16)], rng_v)
        rng_r = jnp.sum(rng_v[...]) * jnp.float32(1.0 / 16.0)

        def scan_sums(i, carry):
            run, cnt = carry
            v = sums_v[pl.ds(i * 16, 16)]
            cs = plsc.cumsum(v) + run
            idxv = iota16 + i * 16
            below = cs < rng_r
            cnt = cnt + jnp.sum(jnp.where(below & (idxv < LASTCHUNK + 1),
                                          1, 0))
            run = run + jnp.sum(v)
            return run, cnt

        _, cnt = lax.fori_loop(0, NCH_PAD // 16, scan_sums,
                               (jnp.float32(0.0), jnp.int32(0)))
        cb2 = jnp.minimum(cnt, LASTCHUNK)

        def bp_body(i, acc):
            v = sums_v[pl.ds(i * 16, 16)]
            idxv = iota16 + i * 16
            return acc + jnp.sum(jnp.where(idxv < cb2, v, 0.0))

        bp = lax.fori_loop(0, NCH_PAD // 16, bp_body, jnp.float32(0.0))
        start = jnp.minimum(cb2 * CHUNK, WSTART_MAX)
        d = cb2 * CHUNK - start
        thr = rng_r - bp

        rg = 8 * (r // 8)
        q = r - rg
        pltpu.sync_copy(p_hbm.at[pl.ds(rg, 8), pl.ds(start, WIN)], win_v)

        def head_body(i, acc):
            v = win_v[q, pl.ds(i * 16, 16)]
            widx = iota16 + i * 16
            return acc + jnp.sum(jnp.where(widx < d, v, 0.0))

        head = lax.fori_loop(0, 512 // 16, head_body, jnp.float32(0.0))
        t = thr + head

        def scan_win(i, carry):
            run2, cnt2 = carry
            v = win_v[q, pl.ds(i * 16, 16)]
            cs = plsc.cumsum(v) + run2
            gl = iota16 + i * 16 + start
            cnt2 = cnt2 + jnp.sum(jnp.where((cs < t) & (gl < VOCAB), 1, 0))
            run2 = run2 + jnp.sum(v)
            return run2, cnt2

        _, cnt2 = lax.fori_loop(0, WIN // 16, scan_win,
                                (jnp.float32(0.0), jnp.int32(0)))
        total = start + cnt2
        out_v[...] = jnp.where(iota16 == 0, total, 0)
        pltpu.sync_copy(out_v, out_hbm.at[pl.ds(r * 16, 16)])


def kernel(p):
    rng = jax.random.uniform(jax.random.key(0), (ROWS,), dtype=jnp.float32)

    s3 = pl.pallas_call(
        _sums_body,
        grid=(NBLK,),
        in_specs=[pl.BlockSpec((ROWS, L), lambda b: (0, b))],
        out_specs=pl.BlockSpec((1, ROWS, CPB), lambda b: (b, 0, 0)),
        out_shape=jax.ShapeDtypeStruct((NBLK, ROWS, CPB), jnp.float32),
    )(p)

    sums = jnp.zeros((ROWS, NCH_PAD), jnp.float32)
    sums = sums.at[:, :NCHUNK].set(
        s3.transpose(1, 0, 2).reshape(ROWS, NCHUNK))
    sums1d = sums.reshape(ROWS * NCH_PAD)
    rng1d = jnp.tile(rng.reshape(ROWS, 1), (1, 16)).reshape(ROWS * 16)

    fin = functools.partial(
        pl.kernel,
        out_type=jax.ShapeDtypeStruct((ROWS * 16,), jnp.int32),
        mesh=plsc.VectorSubcoreMesh(core_axis_name="c", subcore_axis_name="s"),
        scratch_types=[
            pltpu.VMEM((NCH_PAD,), jnp.float32),
            pltpu.VMEM((8, WIN), jnp.float32),
            pltpu.VMEM((16,), jnp.float32),
            pltpu.VMEM((16,), jnp.int32),
        ],
        compiler_params=pltpu.CompilerParams(needs_layout_passes=False),
    )(_sc_finish)
    out16 = fin(sums1d, rng1d, p)

    return jax.lax.stop_gradient(out16.reshape(ROWS, 16)[:, :1])


# speedup vs baseline: 1.6124x; 1.3210x over previous
import functools

import jax
import jax.numpy as jnp
from jax import lax
from jax.experimental import pallas as pl
from jax.experimental.pallas import tpu as pltpu
from jax.experimental.pallas import tpu_sc as plsc

ROWS = 64
VOCAB = 1_000_000
L = 16_384
NBLK = -(-VOCAB // L)
CHUNK = 1_024
CPB = L // CHUNK
NCHUNK = NBLK * CPB
NCH_PAD = 1_024
LASTCHUNK = (VOCAB - 1) // CHUNK
WIN = CHUNK + 128
WSTART_MAX = 998_912
NW = 32
RPW = ROWS // NW


def _sums_body(p_ref, out_ref):
    b = pl.program_id(0)
    x = p_ref[:, :]
    lane = jax.lax.broadcasted_iota(jnp.int32, (ROWS, L), 1) + b * L
    x = jnp.where(lane < VOCAB, x, 0.0)
    parts = [jnp.sum(x[:, c * CHUNK:(c + 1) * CHUNK], axis=1, keepdims=True)
             for c in range(CPB)]
    out_ref[0, :, :] = jnp.concatenate(parts, axis=1)


def _sc_finish(sums_hbm, rng_hbm, p_hbm, out_hbm, sums_v, win_v, rng_v,
               out_v):
    cid = lax.axis_index("c")
    sid = lax.axis_index("s")
    wid = sid * 2 + cid
    iota16 = lax.iota(jnp.int32, 16)

    for k in range(RPW):
        r = wid * RPW + k
        pltpu.sync_copy(sums_hbm.at[pl.ds(r * NCH_PAD, NCH_PAD)], sums_v)
        pltpu.sync_copy(rng_hbm.at[pl.ds(r * 16, 16)], rng_v)
        rng_r = jnp.sum(rng_v[...]) * jnp.float32(1.0 / 16.0)

        def scan_sums(i, carry):
            run, cnt = carry
            v = sums_v[pl.ds(i * 16, 16)]
            cs = plsc.cumsum(v) + run
            idxv = iota16 + i * 16
            below = cs < rng_r
            cnt = cnt + jnp.sum(jnp.where(below & (idxv < LASTCHUNK + 1),
                                          1, 0))
            run = run + jnp.sum(v)
            return run, cnt

        _, cnt = lax.fori_loop(0, NCH_PAD // 16, scan_sums,
                               (jnp.float32(0.0), jnp.int32(0)))
        cb2 = jnp.minimum(cnt, LASTCHUNK)

        def bp_body(i, acc):
            v = sums_v[pl.ds(i * 16, 16)]
            idxv = iota16 + i * 16
            return acc + jnp.sum(jnp.where(idxv < cb2, v, 0.0))

        bp = lax.fori_loop(0, NCH_PAD // 16, bp_body, jnp.float32(0.0))
        start = jnp.minimum(cb2 * CHUNK, WSTART_MAX)
        d = cb2 * CHUNK - start
        thr = rng_r - bp

        rg = 8 * (r // 8)
        q = r - rg
        pltpu.sync_copy(p_hbm.at[pl.ds(rg, 8), pl.ds(start, WIN)], win_v)

        def head_body(i, acc):
            v = win_v[q, pl.ds(i * 16, 16)]
            widx = iota16 + i * 16
            return acc + jnp.sum(jnp.where(widx < d, v, 0.0))

        head = lax.fori_loop(0, 512 // 16, head_body, jnp.float32(0.0))
        t = thr + head

        def scan_win(i, carry):
            run2, cnt2 = carry
            v = win_v[q, pl.ds(i * 16, 16)]
            cs = plsc.cumsum(v) + run2
            gl = iota16 + i * 16 + start
            cnt2 = cnt2 + jnp.sum(jnp.where((cs < t) & (gl < VOCAB), 1, 0))
            run2 = run2 + jnp.sum(v)
            return run2, cnt2

        _, cnt2 = lax.fori_loop(0, WIN // 16, scan_win,
                                (jnp.float32(0.0), jnp.int32(0)))
        total = start + cnt2
        out_v[...] = jnp.where(iota16 == 0, total, 0)
        pltpu.sync_copy(out_v, out_hbm.at[pl.ds(r * 16, 16)])


def kernel(p):
    rng = jax.random.uniform(jax.random.key(0), (ROWS,), dtype=jnp.float32)

    s3 = pl.pallas_call(
        _sums_body,
        grid=(NBLK,),
        in_specs=[pl.BlockSpec((ROWS, L), lambda b: (0, b))],
        out_specs=pl.BlockSpec((1, ROWS, CPB), lambda b: (b, 0, 0)),
        out_shape=jax.ShapeDtypeStruct((NBLK, ROWS, CPB), jnp.float32),
    )(p)

    sums = jnp.zeros((ROWS, NCH_PAD), jnp.float32)
    sums = sums.at[:, :NCHUNK].set(
        s3.transpose(1, 0, 2).reshape(ROWS, NCHUNK))
    sums1d = sums.reshape(ROWS * NCH_PAD)
    rng1d = jnp.tile(rng.reshape(ROWS, 1), (1, 16)).reshape(ROWS * 16)

    fin = functools.partial(
        pl.kernel,
        out_type=jax.ShapeDtypeStruct((ROWS * 16,), jnp.int32),
        mesh=plsc.VectorSubcoreMesh(core_axis_name="c", subcore_axis_name="s"),
        scratch_types=[
            pltpu.VMEM((NCH_PAD,), jnp.float32),
            pltpu.VMEM((8, WIN), jnp.float32),
            pltpu.VMEM((16,), jnp.float32),
            pltpu.VMEM((16,), jnp.int32),
        ],
        compiler_params=pltpu.CompilerParams(needs_layout_passes=False),
    )(_sc_finish)
    out16 = fin(sums1d, rng1d, p)

    return jax.lax.stop_gradient(out16.reshape(ROWS, 16)[:, :1])


# speedup vs baseline: 1.7707x; 1.0982x over previous
import functools

import jax
import jax.numpy as jnp
from jax import lax
from jax.experimental import pallas as pl
from jax.experimental.pallas import tpu as pltpu
from jax.experimental.pallas import tpu_sc as plsc

ROWS = 64
VOCAB = 1_000_000
L = 32_768
NBLK = -(-VOCAB // L)
CHUNK = 1_024
CPB = L // CHUNK
NCHUNK = NBLK * CPB
NCH_PAD = 1_024
LASTCHUNK = (VOCAB - 1) // CHUNK
WIN = CHUNK + 128
WSTART_MAX = 998_912
NW = 32
RPW = ROWS // NW


def _sums_body(p_ref, out_ref):
    b = pl.program_id(0)
    x = p_ref[:, :]
    lane = jax.lax.broadcasted_iota(jnp.int32, (ROWS, L), 1) + b * L
    x = jnp.where(lane < VOCAB, x, 0.0)
    parts = [jnp.sum(x[:, c * CHUNK:(c + 1) * CHUNK], axis=1, keepdims=True)
             for c in range(CPB)]
    out_ref[0, :, :] = jnp.concatenate(parts, axis=1)


def _sc_finish(sums_hbm, rng_hbm, p_hbm, out_hbm, sums_v, win_v, rng_v,
               out_v):
    cid = lax.axis_index("c")
    sid = lax.axis_index("s")
    wid = sid * 2 + cid
    iota16 = lax.iota(jnp.int32, 16)

    for k in range(RPW):
        r = wid * RPW + k
        pltpu.sync_copy(sums_hbm.at[pl.ds(r * NCH_PAD, NCH_PAD)], sums_v)
        pltpu.sync_copy(rng_hbm.at[pl.ds(r * 16, 16)], rng_v)
        rng_r = jnp.sum(rng_v[...]) * jnp.float32(1.0 / 16.0)

        def scan_sums(i, carry):
            run, cnt = carry
            v = sums_v[pl.ds(i * 16, 16)]
            cs = plsc.cumsum(v) + run
            idxv = iota16 + i * 16
            below = cs < rng_r
            cnt = cnt + jnp.sum(jnp.where(below & (idxv < LASTCHUNK + 1),
                                          1, 0))
            run = run + jnp.sum(v)
            return run, cnt

        _, cnt = lax.fori_loop(0, NCH_PAD // 16, scan_sums,
                               (jnp.float32(0.0), jnp.int32(0)))
        cb2 = jnp.minimum(cnt, LASTCHUNK)

        def bp_body(i, acc):
            v = sums_v[pl.ds(i * 16, 16)]
            idxv = iota16 + i * 16
            return acc + jnp.sum(jnp.where(idxv < cb2, v, 0.0))

        bp = lax.fori_loop(0, NCH_PAD // 16, bp_body, jnp.float32(0.0))
        start = jnp.minimum(cb2 * CHUNK, WSTART_MAX)
        d = cb2 * CHUNK - start
        thr = rng_r - bp

        rg = 8 * (r // 8)
        q = r - rg
        pltpu.sync_copy(p_hbm.at[pl.ds(rg, 8), pl.ds(start, WIN)], win_v)

        def head_body(i, acc):
            v = win_v[q, pl.ds(i * 16, 16)]
            widx = iota16 + i * 16
            return acc + jnp.sum(jnp.where(widx < d, v, 0.0))

        head = lax.fori_loop(0, 512 // 16, head_body, jnp.float32(0.0))
        t = thr + head

        def scan_win(i, carry):
            run2, cnt2 = carry
            v = win_v[q, pl.ds(i * 16, 16)]
            cs = plsc.cumsum(v) + run2
            gl = iota16 + i * 16 + start
            cnt2 = cnt2 + jnp.sum(jnp.where((cs < t) & (gl < VOCAB), 1, 0))
            run2 = run2 + jnp.sum(v)
            return run2, cnt2

        _, cnt2 = lax.fori_loop(0, WIN // 16, scan_win,
                                (jnp.float32(0.0), jnp.int32(0)))
        total = start + cnt2
        out_v[...] = jnp.where(iota16 == 0, total, 0)
        pltpu.sync_copy(out_v, out_hbm.at[pl.ds(r * 16, 16)])


def kernel(p):
    rng = jax.random.uniform(jax.random.key(0), (ROWS,), dtype=jnp.float32)

    s3 = pl.pallas_call(
        _sums_body,
        grid=(NBLK,),
        in_specs=[pl.BlockSpec((ROWS, L), lambda b: (0, b))],
        out_specs=pl.BlockSpec((1, ROWS, CPB), lambda b: (b, 0, 0)),
        out_shape=jax.ShapeDtypeStruct((NBLK, ROWS, CPB), jnp.float32),
    )(p)

    sums = jnp.zeros((ROWS, NCH_PAD), jnp.float32)
    sums = sums.at[:, :NCHUNK].set(
        s3.transpose(1, 0, 2).reshape(ROWS, NCHUNK))
    sums1d = sums.reshape(ROWS * NCH_PAD)
    rng1d = jnp.tile(rng.reshape(ROWS, 1), (1, 16)).reshape(ROWS * 16)

    fin = functools.partial(
        pl.kernel,
        out_type=jax.ShapeDtypeStruct((ROWS * 16,), jnp.int32),
        mesh=plsc.VectorSubcoreMesh(core_axis_name="c", subcore_axis_name="s"),
        scratch_types=[
            pltpu.VMEM((NCH_PAD,), jnp.float32),
            pltpu.VMEM((8, WIN), jnp.float32),
            pltpu.VMEM((16,), jnp.float32),
            pltpu.VMEM((16,), jnp.int32),
        ],
        compiler_params=pltpu.CompilerParams(needs_layout_passes=False),
    )(_sc_finish)
    out16 = fin(sums1d, rng1d, p)

    return jax.lax.stop_gradient(out16.reshape(ROWS, 16)[:, :1])


# speedup vs baseline: 1.7809x; 1.0058x over previous
import functools

import jax
import jax.numpy as jnp
from jax import lax
from jax.experimental import pallas as pl
from jax.experimental.pallas import tpu as pltpu
from jax.experimental.pallas import tpu_sc as plsc

ROWS = 64
VOCAB = 1_000_000
L = 65_536
NBLK = -(-VOCAB // L)
CHUNK = 1_024
CPB = L // CHUNK
NCHUNK = NBLK * CPB
NCH_PAD = 1_024
LASTCHUNK = (VOCAB - 1) // CHUNK
WIN = CHUNK + 128
WSTART_MAX = 998_912
NW = 32
RPW = ROWS // NW


def _sums_body(p_ref, out_ref):
    b = pl.program_id(0)
    x = p_ref[:, :]
    lane = jax.lax.broadcasted_iota(jnp.int32, (ROWS, L), 1) + b * L
    x = jnp.where(lane < VOCAB, x, 0.0)
    parts = [jnp.sum(x[:, c * CHUNK:(c + 1) * CHUNK], axis=1, keepdims=True)
             for c in range(CPB)]
    out_ref[0, :, :] = jnp.concatenate(parts, axis=1)


def _sc_finish(sums_hbm, rng_hbm, p_hbm, out_hbm, sums_v, win_v, rng_v,
               out_v):
    cid = lax.axis_index("c")
    sid = lax.axis_index("s")
    wid = sid * 2 + cid
    iota16 = lax.iota(jnp.int32, 16)

    for k in range(RPW):
        r = wid * RPW + k
        pltpu.sync_copy(sums_hbm.at[pl.ds(r * NCH_PAD, NCH_PAD)], sums_v)
        pltpu.sync_copy(rng_hbm.at[pl.ds(r * 16, 16)], rng_v)
        rng_r = jnp.sum(rng_v[...]) * jnp.float32(1.0 / 16.0)

        def scan_sums(i, carry):
            run, cnt = carry
            v = sums_v[pl.ds(i * 16, 16)]
            cs = plsc.cumsum(v) + run
            idxv = iota16 + i * 16
            below = cs < rng_r
            cnt = cnt + jnp.sum(jnp.where(below & (idxv < LASTCHUNK + 1),
                                          1, 0))
            run = run + jnp.sum(v)
            return run, cnt

        _, cnt = lax.fori_loop(0, NCH_PAD // 16, scan_sums,
                               (jnp.float32(0.0), jnp.int32(0)))
        cb2 = jnp.minimum(cnt, LASTCHUNK)

        def bp_body(i, acc):
            v = sums_v[pl.ds(i * 16, 16)]
            idxv = iota16 + i * 16
            return acc + jnp.sum(jnp.where(idxv < cb2, v, 0.0))

        bp = lax.fori_loop(0, NCH_PAD // 16, bp_body, jnp.float32(0.0))
        start = jnp.minimum(cb2 * CHUNK, WSTART_MAX)
        d = cb2 * CHUNK - start
        thr = rng_r - bp

        rg = 8 * (r // 8)
        q = r - rg
        pltpu.sync_copy(p_hbm.at[pl.ds(rg, 8), pl.ds(start, WIN)], win_v)

        def head_body(i, acc):
            v = win_v[q, pl.ds(i * 16, 16)]
            widx = iota16 + i * 16
            return acc + jnp.sum(jnp.where(widx < d, v, 0.0))

        head = lax.fori_loop(0, 512 // 16, head_body, jnp.float32(0.0))
        t = thr + head

        def scan_win(i, carry):
            run2, cnt2 = carry
            v = win_v[q, pl.ds(i * 16, 16)]
            cs = plsc.cumsum(v) + run2
            gl = iota16 + i * 16 + start
            cnt2 = cnt2 + jnp.sum(jnp.where((cs < t) & (gl < VOCAB), 1, 0))
            run2 = run2 + jnp.sum(v)
            return run2, cnt2

        _, cnt2 = lax.fori_loop(0, WIN // 16, scan_win,
                                (jnp.float32(0.0), jnp.int32(0)))
        total = start + cnt2
        out_v[...] = jnp.where(iota16 == 0, total, 0)
        pltpu.sync_copy(out_v, out_hbm.at[pl.ds(r * 16, 16)])


def kernel(p):
    rng = jax.random.uniform(jax.random.key(0), (ROWS,), dtype=jnp.float32)

    s3 = pl.pallas_call(
        _sums_body,
        grid=(NBLK,),
        in_specs=[pl.BlockSpec((ROWS, L), lambda b: (0, b))],
        out_specs=pl.BlockSpec((1, ROWS, CPB), lambda b: (b, 0, 0)),
        out_shape=jax.ShapeDtypeStruct((NBLK, ROWS, CPB), jnp.float32),
    )(p)

    sums = jnp.zeros((ROWS, NCH_PAD), jnp.float32)
    sums = sums.at[:, :NCHUNK].set(
        s3.transpose(1, 0, 2).reshape(ROWS, NCHUNK))
    sums1d = sums.reshape(ROWS * NCH_PAD)
    rng1d = jnp.tile(rng.reshape(ROWS, 1), (1, 16)).reshape(ROWS * 16)

    fin = functools.partial(
        pl.kernel,
        out_type=jax.ShapeDtypeStruct((ROWS * 16,), jnp.int32),
        mesh=plsc.VectorSubcoreMesh(core_axis_name="c", subcore_axis_name="s"),
        scratch_types=[
            pltpu.VMEM((NCH_PAD,), jnp.float32),
            pltpu.VMEM((8, WIN), jnp.float32),
            pltpu.VMEM((16,), jnp.float32),
            pltpu.VMEM((16,), jnp.int32),
        ],
        compiler_params=pltpu.CompilerParams(needs_layout_passes=False),
    )(_sc_finish)
    out16 = fin(sums1d, rng1d, p)

    return jax.lax.stop_gradient(out16.reshape(ROWS, 16)[:, :1])


# speedup vs baseline: 1.7820x; 1.0006x over previous
import functools

import jax
import jax.numpy as jnp
from jax import lax
from jax.experimental import pallas as pl
from jax.experimental.pallas import tpu as pltpu
from jax.experimental.pallas import tpu_sc as plsc

ROWS = 64
VOCAB = 1_000_000
L = 65_536
NBLK = -(-VOCAB // L)
CHUNK = 1_024
CPB = L // CHUNK
NCHUNK = NBLK * CPB
NCH_PAD = 1_024
LASTCHUNK = (VOCAB - 1) // CHUNK
WIN = CHUNK + 128
WSTART_MAX = 998_912
NW = 32
RPW = ROWS // NW


def _sums_body(p_ref, out_ref):
    b = pl.program_id(0)
    x = p_ref[:, :]
    lane = jax.lax.broadcasted_iota(jnp.int32, (ROWS, L), 1) + b * L
    x = jnp.where(lane < VOCAB, x, 0.0)
    parts = [jnp.sum(x[:, c * CHUNK:(c + 1) * CHUNK], axis=1, keepdims=True)
             for c in range(CPB)]
    out_ref[0, :, :] = jnp.concatenate(parts, axis=1)


def _sc_finish(sums_hbm, rng_hbm, p_hbm, out_hbm, sums_v, win_v, rng_v,
               out_v):
    cid = lax.axis_index("c")
    sid = lax.axis_index("s")
    wid = sid * 2 + cid
    iota16 = lax.iota(jnp.int32, 16)

    for k in range(RPW):
        r = wid * RPW + k
        pltpu.sync_copy(sums_hbm.at[pl.ds(r * NCH_PAD, NCH_PAD)], sums_v)
        pltpu.sync_copy(rng_hbm.at[pl.ds(r * 16, 16)], rng_v)
        rng_r = jnp.sum(rng_v[...]) * jnp.float32(1.0 / 16.0)

        def scan_sums(i, carry):
            run, cnt = carry
            v = sums_v[pl.ds(i * 16, 16)]
            cs = plsc.cumsum(v) + run
            idxv = iota16 + i * 16
            below = cs < rng_r
            cnt = cnt + jnp.sum(jnp.where(below & (idxv < LASTCHUNK + 1),
                                          1, 0))
            run = run + jnp.sum(v)
            return run, cnt

        _, cnt = lax.fori_loop(0, NCH_PAD // 16, scan_sums,
                               (jnp.float32(0.0), jnp.int32(0)))
        cb2 = jnp.minimum(cnt, LASTCHUNK)

        def bp_body(i, acc):
            v = sums_v[pl.ds(i * 16, 16)]
            idxv = iota16 + i * 16
            return acc + jnp.sum(jnp.where(idxv < cb2, v, 0.0))

        bp = lax.fori_loop(0, NCH_PAD // 16, bp_body, jnp.float32(0.0))
        start = jnp.minimum(cb2 * CHUNK, WSTART_MAX)
        d = cb2 * CHUNK - start
        thr = rng_r - bp

        rg = 8 * (r // 8)
        q = r - rg
        pltpu.sync_copy(p_hbm.at[pl.ds(rg, 8), pl.ds(start, WIN)], win_v)

        def head_body(i, acc):
            v = win_v[q, pl.ds(i * 16, 16)]
            widx = iota16 + i * 16
            return acc + jnp.sum(jnp.where(widx < d, v, 0.0))

        head = lax.fori_loop(0, 512 // 16, head_body, jnp.float32(0.0))
        t = thr + head

        def scan_win(i, carry):
            run2, cnt2 = carry
            v = win_v[q, pl.ds(i * 16, 16)]
            cs = plsc.cumsum(v) + run2
            gl = iota16 + i * 16 + start
            cnt2 = cnt2 + jnp.sum(jnp.where((cs < t) & (gl < VOCAB), 1, 0))
            run2 = run2 + jnp.sum(v)
            return run2, cnt2

        _, cnt2 = lax.fori_loop(0, WIN // 16, scan_win,
                                (jnp.float32(0.0), jnp.int32(0)))
        total = start + cnt2
        out_v[...] = jnp.where(iota16 == 0, total, 0)
        pltpu.sync_copy(out_v, out_hbm.at[pl.ds(r * 16, 16)])


def kernel(p):
    rng = jax.random.uniform(jax.random.key(0), (ROWS,), dtype=jnp.float32)

    s3 = pl.pallas_call(
        _sums_body,
        grid=(NBLK,),
        in_specs=[pl.BlockSpec((ROWS, L), lambda b: (0, b))],
        out_specs=pl.BlockSpec((1, ROWS, CPB), lambda b: (b, 0, 0)),
        out_shape=jax.ShapeDtypeStruct((NBLK, ROWS, CPB), jnp.float32),
    )(p)

    sums = s3.transpose(1, 0, 2).reshape(ROWS, NCHUNK)
    if NCHUNK < NCH_PAD:
        sums = jnp.pad(sums, ((0, 0), (0, NCH_PAD - NCHUNK)))
    sums1d = sums.reshape(ROWS * NCH_PAD)
    rng1d = jnp.tile(rng.reshape(ROWS, 1), (1, 16)).reshape(ROWS * 16)

    fin = functools.partial(
        pl.kernel,
        out_type=jax.ShapeDtypeStruct((ROWS * 16,), jnp.int32),
        mesh=plsc.VectorSubcoreMesh(core_axis_name="c", subcore_axis_name="s"),
        scratch_types=[
            pltpu.VMEM((NCH_PAD,), jnp.float32),
            pltpu.VMEM((8, WIN), jnp.float32),
            pltpu.VMEM((16,), jnp.float32),
            pltpu.VMEM((16,), jnp.int32),
        ],
        compiler_params=pltpu.CompilerParams(needs_layout_passes=False),
    )(_sc_finish)
    out16 = fin(sums1d, rng1d, p)

    return jax.lax.stop_gradient(out16.reshape(ROWS, 16)[:, :1])
